# Initial kernel scaffold; baseline (speedup 1.0000x reference)
import jax
import jax.numpy as jnp
from jax.experimental import pallas as pl


def kernel(edge_index, edge_weight, mask, eps, W0, Wmu, Wlogvar):
    raise NotImplementedError("write your pallas kernel here")


# speedup vs baseline: 1.1150x; 1.1150x over previous
import functools

import jax
import jax.numpy as jnp
from jax import lax
from jax.experimental import pallas as pl
from jax.experimental.pallas import tpu as pltpu

_N = 10000
_E = 160000
_HID = 256
_LAT = 64

_RB = 400
_BM = 400
_BN = 2000

_INTERPRET = False


def _mm_body(hp_ref, w_ref, omu_ref, olv_ref):
    h = jnp.maximum(hp_ref[...], 0.0)
    r = jnp.dot(h, w_ref[...], preferred_element_type=jnp.float32)
    omu_ref[...] = r[:, :_LAT]
    olv_ref[...] = r[:, _LAT:]


def _enc_matmul(hp, wcat):
    grid = (_N // _RB,)
    return pl.pallas_call(
        _mm_body,
        grid=grid,
        in_specs=[
            pl.BlockSpec((_RB, _HID), lambda i: (i, 0)),
            pl.BlockSpec((_HID, 2 * _LAT), lambda i: (0, 0)),
        ],
        out_specs=[
            pl.BlockSpec((_RB, _LAT), lambda i: (i, 0)),
            pl.BlockSpec((_RB, _LAT), lambda i: (i, 0)),
        ],
        out_shape=[
            jax.ShapeDtypeStruct((_N, _LAT), jnp.float32),
            jax.ShapeDtypeStruct((_N, _LAT), jnp.float32),
        ],
        interpret=_INTERPRET,
    )(hp, wcat)


def _prep_body(zm_ref, zlv_ref, eps_ref, mask_ref, w0_ref, wmu_ref, wlv_ref,
               z_ref, zmask_ref, sums_ref):
    i = pl.program_id(0)
    zm = zm_ref[...]
    zlv = zlv_ref[...]
    z = zm + eps_ref[...] * jnp.exp(0.5 * zlv)
    z_ref[...] = z
    mask = mask_ref[...]
    zmask_ref[...] = mask * z

    kl_part = jnp.sum(zlv - zm * zm - jnp.exp(zlv) + 1.0)
    l2_w0 = jnp.sum(w0_ref[...] * w0_ref[...])
    l2_w = jnp.sum(wmu_ref[...] * wmu_ref[...]) + jnp.sum(wlv_ref[...] * wlv_ref[...])
    msum = jnp.sum(mask)

    lane = lax.broadcasted_iota(jnp.int32, (1, 128), 1)
    vec = (jnp.where(lane == 0, kl_part, 0.0)
           + jnp.where(lane == 1, l2_w0, 0.0)
           + jnp.where((lane == 2) & (i == 0), l2_w, 0.0)
           + jnp.where(lane == 3, msum, 0.0))

    @pl.when(i == 0)
    def _():
        sums_ref[...] = jnp.zeros_like(sums_ref)

    sums_ref[...] += vec


def _prep(z_mean, z_log_var, eps, mask2d, w0, wmu, wlv):
    grid = (_N // _RB,)
    return pl.pallas_call(
        _prep_body,
        grid=grid,
        in_specs=[
            pl.BlockSpec((_RB, _LAT), lambda i: (i, 0)),
            pl.BlockSpec((_RB, _LAT), lambda i: (i, 0)),
            pl.BlockSpec((_RB, _LAT), lambda i: (i, 0)),
            pl.BlockSpec((_RB, 1), lambda i: (i, 0)),
            pl.BlockSpec((_RB, _HID), lambda i: (i, 0)),
            pl.BlockSpec((_HID, _LAT), lambda i: (0, 0)),
            pl.BlockSpec((_HID, _LAT), lambda i: (0, 0)),
        ],
        out_specs=[
            pl.BlockSpec((_RB, _LAT), lambda i: (i, 0)),
            pl.BlockSpec((_RB, _LAT), lambda i: (i, 0)),
            pl.BlockSpec((1, 128), lambda i: (0, 0)),
        ],
        out_shape=[
            jax.ShapeDtypeStruct((_N, _LAT), jnp.float32),
            jax.ShapeDtypeStruct((_N, _LAT), jnp.float32),
            jax.ShapeDtypeStruct((1, 128), jnp.float32),
        ],
        interpret=_INTERPRET,
    )(z_mean, z_log_var, eps, mask2d, w0, wmu, wlv)


def _dec_body(zi_ref, zj_ref, mask_ref, out_ref):
    i = pl.program_id(0)
    j = pl.program_id(1)
    logits = lax.dot_general(zi_ref[...], zj_ref[...],
                             (((1,), (1,)), ((), ())),
                             preferred_element_type=jnp.float32)
    sp = jnp.maximum(logits, 0.0) + jnp.log1p(jnp.exp(-jnp.abs(logits)))
    part = jnp.sum(sp * mask_ref[...])

    @pl.when((i == 0) & (j == 0))
    def _():
        out_ref[0, 0] = 0.0

    out_ref[0, 0] += part


def _decoder_sum(z, mask2d):
    grid = (_N // _BM, _N // _BN)
    return pl.pallas_call(
        _dec_body,
        grid=grid,
        in_specs=[
            pl.BlockSpec((_BM, _LAT), lambda i, j: (i, 0)),
            pl.BlockSpec((_BN, _LAT), lambda i, j: (j, 0)),
            pl.BlockSpec((_BM, 1), lambda i, j: (i, 0)),
        ],
        out_specs=pl.BlockSpec(memory_space=pltpu.SMEM),
        out_shape=jax.ShapeDtypeStruct((1, 1), jnp.float32),
        interpret=_INTERPRET,
    )(z, z, mask2d)[0, 0]


def _segment_sum(vals, row):
    return jax.ops.segment_sum(vals, row, num_segments=_N)


def kernel(edge_index, edge_weight, mask, eps, W0, Wmu, Wlogvar):
    row = edge_index[0]
    col = edge_index[1]
    mask2d = mask.reshape(_N, 1)

    xw = _segment_sum(jnp.take(W0, col, axis=0), row)
    hp = _segment_sum(edge_weight[:, None] * jnp.take(xw, col, axis=0), row)

    wcat = jnp.concatenate([Wmu, Wlogvar], axis=1)
    hw_mu, hw_lv = _enc_matmul(hp, wcat)
    hw = jnp.concatenate([hw_mu, hw_lv], axis=1)
    zcat = _segment_sum(edge_weight[:, None] * jnp.take(hw, col, axis=0), row)
    z_mean = zcat[:, :_LAT]
    z_log_var = zcat[:, _LAT:]

    z, zmask, sums = _prep(z_mean, z_log_var, eps, mask2d, W0, Wmu, Wlogvar)

    t_raw = jnp.sum(jnp.take(zmask, row, axis=0) * jnp.take(z, col, axis=0))

    s1 = _decoder_sum(z, mask2d)

    kl_sum = sums[0, 0]
    l2 = 0.5 * (sums[0, 1] + sums[0, 2])
    msum = sums[0, 3]
    kl = -0.5 * kl_sum / (_N * _LAT)
    masked_ce = (s1 - t_raw) / (_N * _N) / (msum / _N)
    return l2 + masked_ce + kl


# speedup vs baseline: 2.8508x; 2.5568x over previous
import functools

import jax
import jax.numpy as jnp
from jax import lax
from jax.experimental import pallas as pl
from jax.experimental.pallas import tpu as pltpu
from jax.experimental.pallas import tpu_sc as plsc

_N = 10000
_E = 160000
_HID = 256
_LAT = 64

_RB = 400
_BM = 400
_BN = 2000

_NSUB = 16
_CH = 80


_ROW_CHUNKS = [(632 * k, 632) for k in range(15)] + [(632 * 15, _N - 632 * 15)]
_ZROWS = 632


def _make_spmm(width, weighted):
    epw = _E // _NSUB
    nit = epw // _CH
    mesh = plsc.VectorSubcoreMesh(core_axis_name="c", subcore_axis_name="s")

    scratch = [
        pltpu.VMEM((_CH,), jnp.int32),
        pltpu.VMEM((_CH,), jnp.int32),
        pltpu.VMEM((_CH, width), jnp.float32),
        pltpu.VMEM((_CH, 16), jnp.float32),
        pltpu.VMEM_SHARED((_N, width), jnp.float32),
        pltpu.SemaphoreType.DMA,
    ]

    def body(table, col2, rowi, ewr, zeros, out, gi, si, rows, wbuf, acc, sem):
        c = lax.axis_index("c")
        s = lax.axis_index("s")

        for ss, (off, sz) in enumerate(_ROW_CHUNKS):
            @pl.when(s == ss)
            def _(off=off, sz=sz):
                pltpu.sync_copy(zeros.at[pl.ds(0, sz)], acc.at[pl.ds(off, sz)])
        plsc.subcore_barrier()

        def chunk(it, carry):
            base = s * epw + it * _CH
            pltpu.sync_copy(col2.at[pl.ds(c * _E + base, _CH)], gi)
            pltpu.sync_copy(rowi.at[pl.ds(base, _CH)], si)
            pltpu.async_copy(table.at[gi], rows, sem).wait()
            if weighted:
                pltpu.sync_copy(ewr.at[pl.ds(base, _CH)], wbuf)
                for i in range(_CH):
                    w = wbuf[i, :]
                    for j in range(width // 16):
                        rows[i, 16 * j:16 * (j + 1)] = (
                            rows[i, 16 * j:16 * (j + 1)] * w)
            pltpu.sync_copy(rows, acc.at[si], add=True)
            return carry

        lax.fori_loop(0, nit, chunk, jnp.int32(0))
        plsc.subcore_barrier()
        for ss, (off, sz) in enumerate(_ROW_CHUNKS):
            @pl.when(s == ss)
            def _(off=off, sz=sz):
                pltpu.sync_copy(acc.at[pl.ds(off, sz)],
                                out.at[pl.ds(c * _N + off, sz)])

    return functools.partial(
        pl.kernel, body,
        out_type=jax.ShapeDtypeStruct((2 * _N, width), jnp.float32),
        mesh=mesh, scratch_types=scratch)()


def _make_spmm_edge():
    width = 2 * _LAT
    nwork = 2 * _NSUB
    epw = _E // nwork
    ch = 40
    nit = epw // ch
    mesh = plsc.VectorSubcoreMesh(core_axis_name="c", subcore_axis_name="s")

    scratch = [
        pltpu.VMEM((ch,), jnp.int32),
        pltpu.VMEM((ch,), jnp.int32),
        pltpu.VMEM((ch, width), jnp.float32),
        pltpu.VMEM((ch, 16), jnp.float32),
        pltpu.VMEM_SHARED((_N, width), jnp.float32),
        pltpu.SemaphoreType.DMA,
    ]

    def body(table, coli, rowi, ewr, zeros, out, gi, si, rows, wbuf, acc, sem):
        c = lax.axis_index("c")
        s = lax.axis_index("s")

        for ss, (off, sz) in enumerate(_ROW_CHUNKS):
            @pl.when(s == ss)
            def _(off=off, sz=sz):
                pltpu.sync_copy(zeros.at[pl.ds(0, sz)], acc.at[pl.ds(off, sz)])
        plsc.subcore_barrier()

        def chunk(it, carry):
            base = (s * 2 + c) * epw + it * ch
            pltpu.sync_copy(coli.at[pl.ds(base, ch)], gi)
            pltpu.sync_copy(rowi.at[pl.ds(base, ch)], si)
            pltpu.async_copy(table.at[gi], rows, sem).wait()
            pltpu.sync_copy(ewr.at[pl.ds(base, ch)], wbuf)
            for i in range(ch):
                w = wbuf[i, :]
                for j in range(width // 16):
                    rows[i, 16 * j:16 * (j + 1)] = (
                        rows[i, 16 * j:16 * (j + 1)] * w)
            pltpu.sync_copy(rows, acc.at[si], add=True)
            return carry

        lax.fori_loop(0, nit, chunk, jnp.int32(0))
        plsc.subcore_barrier()
        for ss, (off, sz) in enumerate(_ROW_CHUNKS):
            @pl.when(s == ss)
            def _(off=off, sz=sz):
                pltpu.sync_copy(acc.at[pl.ds(off, sz)],
                                out.at[pl.ds(c * _N + off, sz)])

    return functools.partial(
        pl.kernel, body,
        out_type=jax.ShapeDtypeStruct((2 * _N, width), jnp.float32),
        mesh=mesh, scratch_types=scratch)()


def _make_edge_dot():
    nwork = 2 * _NSUB
    epw = _E // nwork
    ch = 40
    nit = epw // ch
    mesh = plsc.VectorSubcoreMesh(core_axis_name="c", subcore_axis_name="s")

    scratch = [
        pltpu.VMEM((ch,), jnp.int32),
        pltpu.VMEM((ch,), jnp.int32),
        pltpu.VMEM((ch, 2 * _LAT), jnp.float32),
        pltpu.VMEM((ch, 2 * _LAT), jnp.float32),
        pltpu.VMEM((16,), jnp.float32),
        pltpu.SemaphoreType.DMA,
    ]

    def body(zfull, rowi, coli, out, gi, si, za, zb, accbuf, sem):
        c = lax.axis_index("c")
        s = lax.axis_index("s")
        wid = s * 2 + c

        def chunk(it, acc):
            base = wid * epw + it * ch
            pltpu.sync_copy(rowi.at[pl.ds(base, ch)], gi)
            pltpu.sync_copy(coli.at[pl.ds(base, ch)], si)
            pltpu.async_copy(zfull.at[gi], za, sem).wait()
            pltpu.async_copy(zfull.at[si], zb, sem).wait()
            for i in range(ch):
                for j in range(_LAT // 16):
                    acc = acc + (za[i, _LAT + 16 * j:_LAT + 16 * (j + 1)]
                                 * zb[i, 16 * j:16 * (j + 1)])
            return acc

        acc = lax.fori_loop(0, nit, chunk, jnp.zeros((16,), jnp.float32))
        accbuf[...] = acc
        pltpu.sync_copy(accbuf, out.at[pl.ds(wid * 16, 16)])

    return functools.partial(
        pl.kernel, body,
        out_type=jax.ShapeDtypeStruct((nwork * 16,), jnp.float32),
        mesh=mesh, scratch_types=scratch)()


def _mm_body(hp_ref, w_ref, o_ref):
    h = jnp.maximum(jnp.concatenate([hp_ref[0], hp_ref[1]], axis=1), 0.0)
    o_ref[...] = jnp.dot(h, w_ref[...], preferred_element_type=jnp.float32)


def _enc_matmul(hp3, wcat):
    grid = (_N // _RB,)
    return pl.pallas_call(
        _mm_body,
        grid=grid,
        in_specs=[
            pl.BlockSpec((2, _RB, _HID // 2), lambda i: (0, i, 0)),
            pl.BlockSpec((_HID, 2 * _LAT), lambda i: (0, 0)),
        ],
        out_specs=pl.BlockSpec((_RB, 2 * _LAT), lambda i: (i, 0)),
        out_shape=jax.ShapeDtypeStruct((_N, 2 * _LAT), jnp.float32),
    )(hp3, wcat)


def _prep_body(zc_ref, eps_ref, mask_ref, w0_ref, wmu_ref, wlv_ref,
               zfull_ref, sums_ref):
    i = pl.program_id(0)
    zcat = zc_ref[0] + zc_ref[1]
    zm = zcat[:, :_LAT]
    zlv = zcat[:, _LAT:]
    z = zm + eps_ref[...] * jnp.exp(0.5 * zlv)
    mask = mask_ref[...]
    zfull_ref[...] = jnp.concatenate([z, mask * z], axis=1)

    kl_part = jnp.sum(zlv - zm * zm - jnp.exp(zlv) + 1.0)
    l2_w0 = jnp.sum(w0_ref[...] * w0_ref[...])
    l2_w = jnp.sum(wmu_ref[...] * wmu_ref[...]) + jnp.sum(wlv_ref[...] * wlv_ref[...])
    msum = jnp.sum(mask)

    lane = lax.broadcasted_iota(jnp.int32, (1, 128), 1)
    vec = (jnp.where(lane == 0, kl_part, 0.0)
           + jnp.where(lane == 1, l2_w0, 0.0)
           + jnp.where((lane == 2) & (i == 0), l2_w, 0.0)
           + jnp.where(lane == 3, msum, 0.0))

    @pl.when(i == 0)
    def _():
        sums_ref[...] = jnp.zeros_like(sums_ref)

    sums_ref[...] += vec


def _prep(zcat3, eps, mask2d, w0, wmu, wlv):
    grid = (_N // _RB,)
    return pl.pallas_call(
        _prep_body,
        grid=grid,
        in_specs=[
            pl.BlockSpec((2, _RB, 2 * _LAT), lambda i: (0, i, 0)),
            pl.BlockSpec((_RB, _LAT), lambda i: (i, 0)),
            pl.BlockSpec((_RB, 1), lambda i: (i, 0)),
            pl.BlockSpec((_RB, _HID), lambda i: (i, 0)),
            pl.BlockSpec((_HID, _LAT), lambda i: (0, 0)),
            pl.BlockSpec((_HID, _LAT), lambda i: (0, 0)),
        ],
        out_specs=[
            pl.BlockSpec((_RB, 2 * _LAT), lambda i: (i, 0)),
            pl.BlockSpec((1, 128), lambda i: (0, 0)),
        ],
        out_shape=[
            jax.ShapeDtypeStruct((_N, 2 * _LAT), jnp.float32),
            jax.ShapeDtypeStruct((1, 128), jnp.float32),
        ],
    )(zcat3, eps, mask2d, w0, wmu, wlv)


def _dec_body(zi_ref, zj_ref, mask_ref, out_ref):
    i = pl.program_id(0)
    j = pl.program_id(1)
    logits = lax.dot_general(zi_ref[:, :_LAT], zj_ref[:, :_LAT],
                             (((1,), (1,)), ((), ())),
                             preferred_element_type=jnp.float32)
    sp = jnp.maximum(logits, 0.0) + jnp.log1p(jnp.exp(-jnp.abs(logits)))
    part = jnp.sum(sp * mask_ref[...])

    @pl.when((i == 0) & (j == 0))
    def _():
        out_ref[0, 0] = 0.0

    out_ref[0, 0] += part


def _decoder_sum(z, mask2d):
    grid = (_N // _BM, _N // _BN)
    return pl.pallas_call(
        _dec_body,
        grid=grid,
        in_specs=[
            pl.BlockSpec((_BM, 2 * _LAT), lambda i, j: (i, 0)),
            pl.BlockSpec((_BN, 2 * _LAT), lambda i, j: (j, 0)),
            pl.BlockSpec((_BM, 1), lambda i, j: (i, 0)),
        ],
        out_specs=pl.BlockSpec(memory_space=pltpu.SMEM),
        out_shape=jax.ShapeDtypeStruct((1, 1), jnp.float32),
    )(z, z, mask2d)


def _fin_body(sums_ref, s1_ref, tp_ref, out_ref):
    t = jnp.sum(tp_ref[...])
    kl_sum = sums_ref[0, 0]
    l2 = 0.5 * (sums_ref[0, 1] + sums_ref[0, 2])
    msum = sums_ref[0, 3]
    kl = -0.5 * kl_sum / (_N * _LAT)
    masked_ce = (s1_ref[0, 0] - t) / (_N * _N) / (msum / _N)
    out_ref[0, 0] = l2 + masked_ce + kl


def _finalize(sums, s1, tpart):
    return pl.pallas_call(
        _fin_body,
        in_specs=[
            pl.BlockSpec((1, 128), lambda: (0, 0)),
            pl.BlockSpec(memory_space=pltpu.SMEM),
            pl.BlockSpec((4, 128), lambda: (0, 0)),
        ],
        out_specs=pl.BlockSpec(memory_space=pltpu.SMEM),
        out_shape=jax.ShapeDtypeStruct((1, 1), jnp.float32),
    )(sums, s1, tpart)


_spmm_plain = _make_spmm(_HID // 2, weighted=False)
_spmm_w256 = _make_spmm(_HID // 2, weighted=True)
_spmm_edge = _make_spmm_edge()
_edge_dot = _make_edge_dot()


def kernel(edge_index, edge_weight, mask, eps, W0, Wmu, Wlogvar):
    row = edge_index[0]
    col = edge_index[1]
    mask2d = mask.reshape(_N, 1)

    col2 = jnp.concatenate([col, col + _N])
    ewr = jnp.tile(edge_weight[:, None], (1, 16))
    zeros128 = jnp.zeros((_ZROWS, _HID // 2), jnp.float32)

    w0t = jnp.concatenate([W0[:, :_HID // 2], W0[:, _HID // 2:]], axis=0)
    xwt = _spmm_plain(w0t, col2, row, ewr, zeros128)
    hpt = _spmm_w256(xwt, col2, row, ewr, zeros128)

    wcat = jnp.concatenate([Wmu, Wlogvar], axis=1)
    hw = _enc_matmul(hpt.reshape(2, _N, _HID // 2), wcat)
    zcat2 = _spmm_edge(hw, col, row, ewr, zeros128)

    zfull, sums = _prep(zcat2.reshape(2, _N, 2 * _LAT), eps, mask2d,
                        W0, Wmu, Wlogvar)

    tpart = _edge_dot(zfull, row, col)
    s1 = _decoder_sum(zfull, mask2d)
    loss = _finalize(sums, s1, tpart.reshape(4, 128))
    return loss[0, 0]


# speedup vs baseline: 4.3767x; 1.5353x over previous
import functools

import jax
import jax.numpy as jnp
from jax import lax
from jax.experimental import pallas as pl
from jax.experimental.pallas import tpu as pltpu
from jax.experimental.pallas import tpu_sc as plsc

_N = 10000
_E = 160000
_HID = 256
_LAT = 64

_RB = 400
_BM = 400
_BN = 2000

_NSUB = 16
_CH = 80

_ROW_CHUNKS = [(632 * k, 632) for k in range(15)] + [(632 * 15, _N - 632 * 15)]
_ZROWS = 632


def _make_spmm(weighted, colsplit):
    width = _HID // 2
    epw = _E // _NSUB if colsplit else _E // (2 * _NSUB)
    nloop = epw // _CH - (1 if epw % _CH == 0 else 0)
    tail = epw - nloop * _CH
    mesh = plsc.VectorSubcoreMesh(core_axis_name="c", subcore_axis_name="s")

    scratch = [
        pltpu.VMEM((_CH,), jnp.int32),
        pltpu.VMEM((_CH,), jnp.int32),
        pltpu.VMEM((tail,), jnp.int32),
        pltpu.VMEM((_CH,), jnp.int32),
        pltpu.VMEM((_CH,), jnp.int32),
        pltpu.VMEM((tail,), jnp.int32),
        pltpu.VMEM((_CH, width), jnp.float32),
        pltpu.VMEM((_CH, width), jnp.float32),
        pltpu.VMEM((_CH, 16), jnp.float32),
        pltpu.VMEM((_CH, 16), jnp.float32),
        pltpu.VMEM_SHARED((_N, width), jnp.float32),
        pltpu.SemaphoreType.DMA,
        pltpu.SemaphoreType.DMA,
        pltpu.SemaphoreType.DMA,
        pltpu.SemaphoreType.DMA,
        pltpu.SemaphoreType.DMA,
        pltpu.SemaphoreType.DMA,
    ]

    def body(table, gidx, rowi, ewr, zeros, out,
             gi0, gi1, git, si0, si1, sit, r0, r1, w0, w1, acc,
             sg0, sg1, ss0, ss1, sw0, sw1):
        c = lax.axis_index("c")
        s = lax.axis_index("s")
        if colsplit:
            goff = c * _E + s * epw
            soff = s * epw
        else:
            goff = (s * 2 + c) * epw
            soff = goff

        gibufs = (gi0, gi1)
        rbufs = (r0, r1)
        sibufs = (si0, si1)
        wbufs = (w0, w1)
        sgs = (sg0, sg1)
        sss = (ss0, ss1)
        sws = (sw0, sw1)

        for cs, (off, sz) in enumerate(_ROW_CHUNKS):
            @pl.when(s == cs)
            def _(off=off, sz=sz):
                pltpu.sync_copy(zeros.at[pl.ds(0, sz)], acc.at[pl.ds(off, sz)])
        plsc.subcore_barrier()

        def idx_descs(k, p):
            descs = [(gidx.at[pl.ds(goff + k * _CH, _CH)], gibufs[p], sgs[p]),
                     (rowi.at[pl.ds(soff + k * _CH, _CH)], sibufs[p], sss[p])]
            if weighted:
                descs.append(
                    (ewr.at[pl.ds(soff + k * _CH, _CH)], wbufs[p], sws[p]))
            return descs

        def issue_idx(k, p):
            for src, dst, sem in idx_descs(k, p):
                pltpu.async_copy(src, dst, sem)

        def wait_idx(k, p):
            for src, dst, sem in idx_descs(k, p):
                pltpu.make_async_copy(src, dst, sem).wait()

        def gather_desc(p):
            return (table.at[gibufs[p]], rbufs[p], sgs[p])

        def scale(p, n):
            rb = rbufs[p]
            wb = wbufs[p]
            for i in range(n):
                w = wb[i, :]
                for j in range(width // 16):
                    rb[i, 16 * j:16 * (j + 1)] = rb[i, 16 * j:16 * (j + 1)] * w

        def step(k, p):
            src, dst, sem = gather_desc(p)
            pltpu.make_async_copy(src, dst, sem).wait()

            @pl.when(k + 1 < nloop)
            def _():
                wait_idx(k + 1, 1 - p)
                s2, d2, m2 = gather_desc(1 - p)
                pltpu.async_copy(s2, d2, m2)

            if weighted:
                scale(p, _CH)
            pltpu.sync_copy(rbufs[p], acc.at[sibufs[p]], add=True)

            @pl.when(k + 2 < nloop)
            def _():
                issue_idx(k + 2, p)

        issue_idx(0, 0)
        issue_idx(1, 1)
        wait_idx(0, 0)
        src0, dst0, sem0 = gather_desc(0)
        pltpu.async_copy(src0, dst0, sem0)

        def it(k, carry):
            @pl.when(k % 2 == 0)
            def _():
                step(k, 0)

            @pl.when(k % 2 == 1)
            def _():
                step(k, 1)

            return carry

        lax.fori_loop(0, nloop, it, jnp.int32(0))

        p = nloop % 2
        tdescs = [(gidx.at[pl.ds(goff + nloop * _CH, tail)], git, sgs[p]),
                  (rowi.at[pl.ds(soff + nloop * _CH, tail)], sit, sss[p])]
        if weighted:
            tdescs.append((ewr.at[pl.ds(soff + nloop * _CH, tail)],
                           wbufs[p].at[pl.ds(0, tail)], sws[p]))
        for src, dst, sem in tdescs:
            pltpu.async_copy(src, dst, sem)
        for src, dst, sem in tdescs:
            pltpu.make_async_copy(src, dst, sem).wait()
        pltpu.async_copy(table.at[git], rbufs[p].at[pl.ds(0, tail)], sgs[p])
        pltpu.make_async_copy(table.at[git], rbufs[p].at[pl.ds(0, tail)],
                              sgs[p]).wait()
        if weighted:
            scale(p, tail)
        pltpu.sync_copy(rbufs[p].at[pl.ds(0, tail)], acc.at[sit], add=True)

        plsc.subcore_barrier()
        for cs, (off, sz) in enumerate(_ROW_CHUNKS):
            @pl.when(s == cs)
            def _(off=off, sz=sz):
                pltpu.sync_copy(acc.at[pl.ds(off, sz)],
                                out.at[pl.ds(c * _N + off, sz)])

    return functools.partial(
        pl.kernel, body,
        out_type=jax.ShapeDtypeStruct((2 * _N, width), jnp.float32),
        mesh=mesh, scratch_types=scratch)()


def _make_edge_dot():
    nwork = 2 * _NSUB
    epw = _E // nwork
    nloop = epw // _CH - (1 if epw % _CH == 0 else 0)
    tail = epw - nloop * _CH
    mesh = plsc.VectorSubcoreMesh(core_axis_name="c", subcore_axis_name="s")

    scratch = [
        pltpu.VMEM((epw,), jnp.int32),
        pltpu.VMEM((epw,), jnp.int32),
        pltpu.VMEM((_CH, 2 * _LAT), jnp.float32),
        pltpu.VMEM((_CH, 2 * _LAT), jnp.float32),
        pltpu.VMEM((_CH, 2 * _LAT), jnp.float32),
        pltpu.VMEM((_CH, 2 * _LAT), jnp.float32),
        pltpu.VMEM((16,), jnp.float32),
        pltpu.SemaphoreType.DMA,
        pltpu.SemaphoreType.DMA,
        pltpu.SemaphoreType.DMA,
        pltpu.SemaphoreType.DMA,
    ]

    def body(zfull, rowi, coli, out,
             gr, gc, za0, za1, zb0, zb1, accv, sa0, sa1, sb0, sb1):
        c = lax.axis_index("c")
        s = lax.axis_index("s")
        wid = s * 2 + c
        off = wid * epw

        zas = (za0, za1)
        zbs = (zb0, zb1)
        sas = (sa0, sa1)
        sbs = (sb0, sb1)

        pltpu.sync_copy(rowi.at[pl.ds(off, epw)], gr)
        pltpu.sync_copy(coli.at[pl.ds(off, epw)], gc)
        accv[...] = jnp.zeros((16,), jnp.float32)

        def gather_descs(k, p, n):
            return [
                (zfull.at[gr.at[pl.ds(k * _CH, n)]],
                 zas[p].at[pl.ds(0, n)], sas[p]),
                (zfull.at[gc.at[pl.ds(k * _CH, n)]],
                 zbs[p].at[pl.ds(0, n)], sbs[p]),
            ]

        def issue(k, p, n):
            for src, dst, sem in gather_descs(k, p, n):
                pltpu.async_copy(src, dst, sem)

        def wait(k, p, n):
            for src, dst, sem in gather_descs(k, p, n):
                pltpu.make_async_copy(src, dst, sem).wait()

        def accum(p, n):
            parts = [jnp.zeros((16,), jnp.float32) for _ in range(4)]
            for i in range(n):
                for j in range(_LAT // 16):
                    parts[j] = parts[j] + (
                        zas[p][i, _LAT + 16 * j:_LAT + 16 * (j + 1)]
                        * zbs[p][i, 16 * j:16 * (j + 1)])
            accv[...] = accv[...] + ((parts[0] + parts[1])
                                     + (parts[2] + parts[3]))

        def step(k, p):
            @pl.when(k + 1 < nloop)
            def _():
                issue(k + 1, 1 - p, _CH)

            wait(k, p, _CH)
            accum(p, _CH)

        issue(0, 0, _CH)

        def it(k, carry):
            @pl.when(k % 2 == 0)
            def _():
                step(k, 0)

            @pl.when(k % 2 == 1)
            def _():
                step(k, 1)

            return carry

        lax.fori_loop(0, nloop, it, jnp.int32(0))

        p = nloop % 2
        issue(nloop, p, tail)
        wait(nloop, p, tail)
        accum(p, tail)

        pltpu.sync_copy(accv, out.at[pl.ds(wid * 16, 16)])

    return functools.partial(
        pl.kernel, body,
        out_type=jax.ShapeDtypeStruct((nwork * 16,), jnp.float32),
        mesh=mesh, scratch_types=scratch)()


def _mm_body(hp_ref, w_ref, o_ref):
    h = jnp.maximum(jnp.concatenate([hp_ref[0], hp_ref[1]], axis=1), 0.0)
    o_ref[...] = jnp.dot(h, w_ref[...], preferred_element_type=jnp.float32)


def _enc_matmul(hp3, wcat):
    grid = (_N // _RB,)
    return pl.pallas_call(
        _mm_body,
        grid=grid,
        in_specs=[
            pl.BlockSpec((2, _RB, _HID // 2), lambda i: (0, i, 0)),
            pl.BlockSpec((_HID, 2 * _LAT), lambda i: (0, 0)),
        ],
        out_specs=pl.BlockSpec((_RB, 2 * _LAT), lambda i: (i, 0)),
        out_shape=jax.ShapeDtypeStruct((_N, 2 * _LAT), jnp.float32),
    )(hp3, wcat)


def _prep_body(zc_ref, eps_ref, mask_ref, w0_ref, wmu_ref, wlv_ref,
               zfull_ref, sums_ref):
    i = pl.program_id(0)
    zcat = zc_ref[0] + zc_ref[1]
    zm = zcat[:, :_LAT]
    zlv = zcat[:, _LAT:]
    z = zm + eps_ref[...] * jnp.exp(0.5 * zlv)
    mask = mask_ref[...]
    zfull_ref[...] = jnp.concatenate([z, mask * z], axis=1)

    kl_part = jnp.sum(zlv - zm * zm - jnp.exp(zlv) + 1.0)
    l2_w0 = jnp.sum(w0_ref[...] * w0_ref[...])
    l2_w = jnp.sum(wmu_ref[...] * wmu_ref[...]) + jnp.sum(wlv_ref[...] * wlv_ref[...])
    msum = jnp.sum(mask)

    lane = lax.broadcasted_iota(jnp.int32, (1, 128), 1)
    vec = (jnp.where(lane == 0, kl_part, 0.0)
           + jnp.where(lane == 1, l2_w0, 0.0)
           + jnp.where((lane == 2) & (i == 0), l2_w, 0.0)
           + jnp.where(lane == 3, msum, 0.0))

    @pl.when(i == 0)
    def _():
        sums_ref[...] = jnp.zeros_like(sums_ref)

    sums_ref[...] += vec


def _prep(zcat3, eps, mask2d, w0, wmu, wlv):
    grid = (_N // _RB,)
    return pl.pallas_call(
        _prep_body,
        grid=grid,
        in_specs=[
            pl.BlockSpec((2, _RB, 2 * _LAT), lambda i: (0, i, 0)),
            pl.BlockSpec((_RB, _LAT), lambda i: (i, 0)),
            pl.BlockSpec((_RB, 1), lambda i: (i, 0)),
            pl.BlockSpec((_RB, _HID), lambda i: (i, 0)),
            pl.BlockSpec((_HID, _LAT), lambda i: (0, 0)),
            pl.BlockSpec((_HID, _LAT), lambda i: (0, 0)),
        ],
        out_specs=[
            pl.BlockSpec((_RB, 2 * _LAT), lambda i: (i, 0)),
            pl.BlockSpec((1, 128), lambda i: (0, 0)),
        ],
        out_shape=[
            jax.ShapeDtypeStruct((_N, 2 * _LAT), jnp.float32),
            jax.ShapeDtypeStruct((1, 128), jnp.float32),
        ],
    )(zcat3, eps, mask2d, w0, wmu, wlv)


def _dec_body(zi_ref, zj_ref, mask_ref, out_ref):
    i = pl.program_id(0)
    j = pl.program_id(1)
    logits = lax.dot_general(zi_ref[:, :_LAT], zj_ref[:, :_LAT],
                             (((1,), (1,)), ((), ())),
                             preferred_element_type=jnp.float32)
    sp = jnp.maximum(logits, 0.0) + jnp.log1p(jnp.exp(-jnp.abs(logits)))
    part = jnp.sum(sp * mask_ref[...])

    @pl.when((i == 0) & (j == 0))
    def _():
        out_ref[0, 0] = 0.0

    out_ref[0, 0] += part


def _decoder_sum(z, mask2d):
    grid = (_N // _BM, _N // _BN)
    return pl.pallas_call(
        _dec_body,
        grid=grid,
        in_specs=[
            pl.BlockSpec((_BM, 2 * _LAT), lambda i, j: (i, 0)),
            pl.BlockSpec((_BN, 2 * _LAT), lambda i, j: (j, 0)),
            pl.BlockSpec((_BM, 1), lambda i, j: (i, 0)),
        ],
        out_specs=pl.BlockSpec(memory_space=pltpu.SMEM),
        out_shape=jax.ShapeDtypeStruct((1, 1), jnp.float32),
    )(z, z, mask2d)


def _fin_body(sums_ref, s1_ref, tp_ref, out_ref):
    t = jnp.sum(tp_ref[...])
    kl_sum = sums_ref[0, 0]
    l2 = 0.5 * (sums_ref[0, 1] + sums_ref[0, 2])
    msum = sums_ref[0, 3]
    kl = -0.5 * kl_sum / (_N * _LAT)
    masked_ce = (s1_ref[0, 0] - t) / (_N * _N) / (msum / _N)
    out_ref[0, 0] = l2 + masked_ce + kl


def _finalize(sums, s1, tpart):
    return pl.pallas_call(
        _fin_body,
        in_specs=[
            pl.BlockSpec((1, 128), lambda: (0, 0)),
            pl.BlockSpec(memory_space=pltpu.SMEM),
            pl.BlockSpec((4, 128), lambda: (0, 0)),
        ],
        out_specs=pl.BlockSpec(memory_space=pltpu.SMEM),
        out_shape=jax.ShapeDtypeStruct((1, 1), jnp.float32),
    )(sums, s1, tpart)


_spmm_plain = _make_spmm(weighted=False, colsplit=True)
_spmm_w256 = _make_spmm(weighted=True, colsplit=True)
_spmm_edge = _make_spmm(weighted=True, colsplit=False)
_edge_dot = _make_edge_dot()


def kernel(edge_index, edge_weight, mask, eps, W0, Wmu, Wlogvar):
    row = edge_index[0]
    col = edge_index[1]
    mask2d = mask.reshape(_N, 1)

    col2 = jnp.concatenate([col, col + _N])
    ewr = jnp.tile(edge_weight[:, None], (1, 16))
    zeros128 = jnp.zeros((_ZROWS, _HID // 2), jnp.float32)

    w0t = jnp.concatenate([W0[:, :_HID // 2], W0[:, _HID // 2:]], axis=0)
    xwt = _spmm_plain(w0t, col2, row, ewr, zeros128)
    hpt = _spmm_w256(xwt, col2, row, ewr, zeros128)

    wcat = jnp.concatenate([Wmu, Wlogvar], axis=1)
    hw = _enc_matmul(hpt.reshape(2, _N, _HID // 2), wcat)
    zcat2 = _spmm_edge(hw, col, row, ewr, zeros128)

    zfull, sums = _prep(zcat2.reshape(2, _N, 2 * _LAT), eps, mask2d,
                        W0, Wmu, Wlogvar)

    tpart = _edge_dot(zfull, row, col)
    s1 = _decoder_sum(zfull, mask2d)
    loss = _finalize(sums, s1, tpart.reshape(4, 128))
    return loss[0, 0]


# speedup vs baseline: 4.5234x; 1.0335x over previous
import functools

import jax
import jax.numpy as jnp
from jax import lax
from jax.experimental import pallas as pl
from jax.experimental.pallas import tpu as pltpu
from jax.experimental.pallas import tpu_sc as plsc

_N = 10000
_E = 160000
_HID = 256
_LAT = 64

_RB = 400
_BM = 400
_BN = 2000

_NSUB = 16
_CH = 80

_ROW_CHUNKS = [(632 * k, 632) for k in range(15)] + [(632 * 15, _N - 632 * 15)]
_ZROWS = 632


def _make_spmm(weighted, colsplit):
    width = _HID // 2
    epw = _E // _NSUB if colsplit else _E // (2 * _NSUB)
    nloop = epw // _CH - (1 if epw % _CH == 0 else 0)
    tail = epw - nloop * _CH
    mesh = plsc.VectorSubcoreMesh(core_axis_name="c", subcore_axis_name="s")

    half = _CH // 2
    scratch = [
        pltpu.VMEM((_CH,), jnp.int32),
        pltpu.VMEM((_CH,), jnp.int32),
        pltpu.VMEM((tail,), jnp.int32),
        pltpu.VMEM((half,), jnp.int32),
        pltpu.VMEM((half,), jnp.int32),
        pltpu.VMEM((half,), jnp.int32),
        pltpu.VMEM((half,), jnp.int32),
        pltpu.VMEM((_CH, width), jnp.float32),
        pltpu.VMEM((_CH, width), jnp.float32),
        pltpu.VMEM((_CH, 16), jnp.float32),
        pltpu.VMEM((_CH, 16), jnp.float32),
        pltpu.VMEM_SHARED((_N, width), jnp.float32),
        pltpu.SemaphoreType.DMA,
        pltpu.SemaphoreType.DMA,
        pltpu.SemaphoreType.DMA,
        pltpu.SemaphoreType.DMA,
        pltpu.SemaphoreType.DMA,
        pltpu.SemaphoreType.DMA,
        pltpu.SemaphoreType.DMA,
        pltpu.SemaphoreType.DMA,
    ]

    def body(table, gidx, rowi, ewr, zeros, out,
             gi0, gi1, git, si0a, si0b, si1a, si1b, r0, r1, w0, w1, acc,
             sg0, sg1, ss0, ss1, sw0, sw1, sc0, sc1):
        c = lax.axis_index("c")
        s = lax.axis_index("s")
        if colsplit:
            goff = c * _E + s * epw
            soff = s * epw
        else:
            goff = (s * 2 + c) * epw
            soff = goff

        gibufs = (gi0, gi1)
        rbufs = (r0, r1)
        sibufs = ((si0a, si0b), (si1a, si1b))
        wbufs = (w0, w1)
        sgs = (sg0, sg1)
        sss = (ss0, ss1)
        sws = (sw0, sw1)
        scs = (sc0, sc1)

        for cs, (off, sz) in enumerate(_ROW_CHUNKS):
            @pl.when(s == cs)
            def _(off=off, sz=sz):
                pltpu.sync_copy(zeros.at[pl.ds(0, sz)], acc.at[pl.ds(off, sz)])
        plsc.subcore_barrier()

        def idx_descs(k, p):
            descs = [(gidx.at[pl.ds(goff + k * _CH, _CH)], gibufs[p], sgs[p]),
                     (rowi.at[pl.ds(soff + k * _CH, half)],
                      sibufs[p][0], sss[p]),
                     (rowi.at[pl.ds(soff + k * _CH + half, half)],
                      sibufs[p][1], sss[p])]
            if weighted:
                descs.append(
                    (ewr.at[pl.ds(soff + k * _CH, _CH)], wbufs[p], sws[p]))
            return descs

        def issue_idx(k, p):
            for src, dst, sem in idx_descs(k, p):
                pltpu.async_copy(src, dst, sem)

        def wait_idx(k, p):
            for src, dst, sem in idx_descs(k, p):
                pltpu.make_async_copy(src, dst, sem).wait()

        def gather_desc(p):
            return (table.at[gibufs[p]], rbufs[p], sgs[p])

        def scale(p, lo, hi):
            rb = rbufs[p]
            wb = wbufs[p]
            for i in range(lo, hi):
                w = wb[i, :]
                for j in range(width // 16):
                    rb[i, 16 * j:16 * (j + 1)] = rb[i, 16 * j:16 * (j + 1)] * w

        def step(k, p):
            src, dst, sem = gather_desc(p)
            pltpu.make_async_copy(src, dst, sem).wait()

            @pl.when(k + 1 < nloop)
            def _():
                wait_idx(k + 1, 1 - p)
                s2, d2, m2 = gather_desc(1 - p)
                pltpu.async_copy(s2, d2, m2)

            if weighted:
                scale(p, 0, half)
                pltpu.async_copy(rbufs[p].at[pl.ds(0, half)],
                                 acc.at[sibufs[p][0]], scs[p], add=True)
                scale(p, half, _CH)
                pltpu.sync_copy(rbufs[p].at[pl.ds(half, half)],
                                acc.at[sibufs[p][1]], add=True)
                pltpu.make_async_copy(rbufs[p].at[pl.ds(0, half)],
                                      acc.at[sibufs[p][0]], scs[p]).wait()
            else:
                pltpu.async_copy(rbufs[p].at[pl.ds(0, half)],
                                 acc.at[sibufs[p][0]], scs[p], add=True)
                pltpu.sync_copy(rbufs[p].at[pl.ds(half, half)],
                                acc.at[sibufs[p][1]], add=True)
                pltpu.make_async_copy(rbufs[p].at[pl.ds(0, half)],
                                      acc.at[sibufs[p][0]], scs[p]).wait()

            @pl.when(k + 2 < nloop)
            def _():
                issue_idx(k + 2, p)

        issue_idx(0, 0)
        issue_idx(1, 1)
        wait_idx(0, 0)
        src0, dst0, sem0 = gather_desc(0)
        pltpu.async_copy(src0, dst0, sem0)

        def it(k, carry):
            @pl.when(k % 2 == 0)
            def _():
                step(k, 0)

            @pl.when(k % 2 == 1)
            def _():
                step(k, 1)

            return carry

        lax.fori_loop(0, nloop, it, jnp.int32(0))

        p = nloop % 2
        nh = tail // half
        tdescs = [(gidx.at[pl.ds(goff + nloop * _CH, tail)], git, sgs[p])]
        for h in range(nh):
            tdescs.append(
                (rowi.at[pl.ds(soff + nloop * _CH + h * half, half)],
                 sibufs[p][h], sss[p]))
        if weighted:
            tdescs.append((ewr.at[pl.ds(soff + nloop * _CH, tail)],
                           wbufs[p].at[pl.ds(0, tail)], sws[p]))
        for src, dst, sem in tdescs:
            pltpu.async_copy(src, dst, sem)
        for src, dst, sem in tdescs:
            pltpu.make_async_copy(src, dst, sem).wait()
        pltpu.async_copy(table.at[git], rbufs[p].at[pl.ds(0, tail)], sgs[p])
        pltpu.make_async_copy(table.at[git], rbufs[p].at[pl.ds(0, tail)],
                              sgs[p]).wait()
        if weighted:
            scale(p, 0, tail)
        for h in range(nh):
            pltpu.sync_copy(rbufs[p].at[pl.ds(h * half, half)],
                            acc.at[sibufs[p][h]], add=True)

        plsc.subcore_barrier()
        for cs, (off, sz) in enumerate(_ROW_CHUNKS):
            @pl.when(s == cs)
            def _(off=off, sz=sz):
                pltpu.sync_copy(acc.at[pl.ds(off, sz)],
                                out.at[pl.ds(c * _N + off, sz)])

    return functools.partial(
        pl.kernel, body,
        out_type=jax.ShapeDtypeStruct((2 * _N, width), jnp.float32),
        mesh=mesh, scratch_types=scratch)()


def _make_edge_dot():
    nwork = 2 * _NSUB
    epw = _E // nwork
    nloop = epw // _CH - (1 if epw % _CH == 0 else 0)
    tail = epw - nloop * _CH
    mesh = plsc.VectorSubcoreMesh(core_axis_name="c", subcore_axis_name="s")

    scratch = [
        pltpu.VMEM((epw,), jnp.int32),
        pltpu.VMEM((epw,), jnp.int32),
        pltpu.VMEM((_CH, 2 * _LAT), jnp.float32),
        pltpu.VMEM((_CH, 2 * _LAT), jnp.float32),
        pltpu.VMEM((_CH, 2 * _LAT), jnp.float32),
        pltpu.VMEM((_CH, 2 * _LAT), jnp.float32),
        pltpu.VMEM((16,), jnp.float32),
        pltpu.SemaphoreType.DMA,
        pltpu.SemaphoreType.DMA,
        pltpu.SemaphoreType.DMA,
        pltpu.SemaphoreType.DMA,
    ]

    def body(zfull, rowi, coli, out,
             gr, gc, za0, za1, zb0, zb1, accv, sa0, sa1, sb0, sb1):
        c = lax.axis_index("c")
        s = lax.axis_index("s")
        wid = s * 2 + c
        off = wid * epw

        zas = (za0, za1)
        zbs = (zb0, zb1)
        sas = (sa0, sa1)
        sbs = (sb0, sb1)

        pltpu.sync_copy(rowi.at[pl.ds(off, epw)], gr)
        pltpu.sync_copy(coli.at[pl.ds(off, epw)], gc)
        accv[...] = jnp.zeros((16,), jnp.float32)

        def gather_descs(k, p, n):
            return [
                (zfull.at[gr.at[pl.ds(k * _CH, n)]],
                 zas[p].at[pl.ds(0, n)], sas[p]),
                (zfull.at[gc.at[pl.ds(k * _CH, n)]],
                 zbs[p].at[pl.ds(0, n)], sbs[p]),
            ]

        def issue(k, p, n):
            for src, dst, sem in gather_descs(k, p, n):
                pltpu.async_copy(src, dst, sem)

        def wait(k, p, n):
            for src, dst, sem in gather_descs(k, p, n):
                pltpu.make_async_copy(src, dst, sem).wait()

        def accum(p, n):
            parts = [jnp.zeros((16,), jnp.float32) for _ in range(4)]
            for i in range(n):
                for j in range(_LAT // 16):
                    parts[j] = parts[j] + (
                        zas[p][i, _LAT + 16 * j:_LAT + 16 * (j + 1)]
                        * zbs[p][i, 16 * j:16 * (j + 1)])
            accv[...] = accv[...] + ((parts[0] + parts[1])
                                     + (parts[2] + parts[3]))

        def step(k, p):
            @pl.when(k + 1 < nloop)
            def _():
                issue(k + 1, 1 - p, _CH)

            wait(k, p, _CH)
            accum(p, _CH)

        issue(0, 0, _CH)

        def it(k, carry):
            @pl.when(k % 2 == 0)
            def _():
                step(k, 0)

            @pl.when(k % 2 == 1)
            def _():
                step(k, 1)

            return carry

        lax.fori_loop(0, nloop, it, jnp.int32(0))

        p = nloop % 2
        issue(nloop, p, tail)
        wait(nloop, p, tail)
        accum(p, tail)

        pltpu.sync_copy(accv, out.at[pl.ds(wid * 16, 16)])

    return functools.partial(
        pl.kernel, body,
        out_type=jax.ShapeDtypeStruct((nwork * 16,), jnp.float32),
        mesh=mesh, scratch_types=scratch)()


def _mm_body(hp_ref, w_ref, o_ref):
    h = jnp.maximum(jnp.concatenate([hp_ref[0], hp_ref[1]], axis=1), 0.0)
    o_ref[...] = jnp.dot(h, w_ref[...], preferred_element_type=jnp.float32)


def _enc_matmul(hp3, wcat):
    grid = (_N // _RB,)
    return pl.pallas_call(
        _mm_body,
        grid=grid,
        in_specs=[
            pl.BlockSpec((2, _RB, _HID // 2), lambda i: (0, i, 0)),
            pl.BlockSpec((_HID, 2 * _LAT), lambda i: (0, 0)),
        ],
        out_specs=pl.BlockSpec((_RB, 2 * _LAT), lambda i: (i, 0)),
        out_shape=jax.ShapeDtypeStruct((_N, 2 * _LAT), jnp.float32),
    )(hp3, wcat)


def _prep_body(zc_ref, eps_ref, mask_ref, w0_ref, wmu_ref, wlv_ref,
               zfull_ref, sums_ref):
    i = pl.program_id(0)
    zcat = zc_ref[0] + zc_ref[1]
    zm = zcat[:, :_LAT]
    zlv = zcat[:, _LAT:]
    z = zm + eps_ref[...] * jnp.exp(0.5 * zlv)
    mask = mask_ref[...]
    zfull_ref[...] = jnp.concatenate([z, mask * z], axis=1)

    kl_part = jnp.sum(zlv - zm * zm - jnp.exp(zlv) + 1.0)
    l2_w0 = jnp.sum(w0_ref[...] * w0_ref[...])
    l2_w = jnp.sum(wmu_ref[...] * wmu_ref[...]) + jnp.sum(wlv_ref[...] * wlv_ref[...])
    msum = jnp.sum(mask)

    lane = lax.broadcasted_iota(jnp.int32, (1, 128), 1)
    vec = (jnp.where(lane == 0, kl_part, 0.0)
           + jnp.where(lane == 1, l2_w0, 0.0)
           + jnp.where((lane == 2) & (i == 0), l2_w, 0.0)
           + jnp.where(lane == 3, msum, 0.0))

    @pl.when(i == 0)
    def _():
        sums_ref[...] = jnp.zeros_like(sums_ref)

    sums_ref[...] += vec


def _prep(zcat3, eps, mask2d, w0, wmu, wlv):
    grid = (_N // _RB,)
    return pl.pallas_call(
        _prep_body,
        grid=grid,
        in_specs=[
            pl.BlockSpec((2, _RB, 2 * _LAT), lambda i: (0, i, 0)),
            pl.BlockSpec((_RB, _LAT), lambda i: (i, 0)),
            pl.BlockSpec((_RB, 1), lambda i: (i, 0)),
            pl.BlockSpec((_RB, _HID), lambda i: (i, 0)),
            pl.BlockSpec((_HID, _LAT), lambda i: (0, 0)),
            pl.BlockSpec((_HID, _LAT), lambda i: (0, 0)),
        ],
        out_specs=[
            pl.BlockSpec((_RB, 2 * _LAT), lambda i: (i, 0)),
            pl.BlockSpec((1, 128), lambda i: (0, 0)),
        ],
        out_shape=[
            jax.ShapeDtypeStruct((_N, 2 * _LAT), jnp.float32),
            jax.ShapeDtypeStruct((1, 128), jnp.float32),
        ],
    )(zcat3, eps, mask2d, w0, wmu, wlv)


def _dec_body(zi_ref, zj_ref, mask_ref, out_ref):
    i = pl.program_id(0)
    j = pl.program_id(1)
    logits = lax.dot_general(zi_ref[:, :_LAT], zj_ref[:, :_LAT],
                             (((1,), (1,)), ((), ())),
                             preferred_element_type=jnp.float32)
    sp = jnp.maximum(logits, 0.0) + jnp.log1p(jnp.exp(-jnp.abs(logits)))
    part = jnp.sum(sp * mask_ref[...])

    @pl.when((i == 0) & (j == 0))
    def _():
        out_ref[0, 0] = 0.0

    out_ref[0, 0] += part


def _decoder_sum(z, mask2d):
    grid = (_N // _BM, _N // _BN)
    return pl.pallas_call(
        _dec_body,
        grid=grid,
        in_specs=[
            pl.BlockSpec((_BM, 2 * _LAT), lambda i, j: (i, 0)),
            pl.BlockSpec((_BN, 2 * _LAT), lambda i, j: (j, 0)),
            pl.BlockSpec((_BM, 1), lambda i, j: (i, 0)),
        ],
        out_specs=pl.BlockSpec(memory_space=pltpu.SMEM),
        out_shape=jax.ShapeDtypeStruct((1, 1), jnp.float32),
    )(z, z, mask2d)


def _fin_body(sums_ref, s1_ref, tp_ref, out_ref):
    t = jnp.sum(tp_ref[...])
    kl_sum = sums_ref[0, 0]
    l2 = 0.5 * (sums_ref[0, 1] + sums_ref[0, 2])
    msum = sums_ref[0, 3]
    kl = -0.5 * kl_sum / (_N * _LAT)
    masked_ce = (s1_ref[0, 0] - t) / (_N * _N) / (msum / _N)
    out_ref[0, 0] = l2 + masked_ce + kl


def _finalize(sums, s1, tpart):
    return pl.pallas_call(
        _fin_body,
        in_specs=[
            pl.BlockSpec((1, 128), lambda: (0, 0)),
            pl.BlockSpec(memory_space=pltpu.SMEM),
            pl.BlockSpec((4, 128), lambda: (0, 0)),
        ],
        out_specs=pl.BlockSpec(memory_space=pltpu.SMEM),
        out_shape=jax.ShapeDtypeStruct((1, 1), jnp.float32),
    )(sums, s1, tpart)


_spmm_plain = _make_spmm(weighted=False, colsplit=True)
_spmm_w256 = _make_spmm(weighted=True, colsplit=True)
_spmm_edge = _make_spmm(weighted=True, colsplit=False)
_edge_dot = _make_edge_dot()


def kernel(edge_index, edge_weight, mask, eps, W0, Wmu, Wlogvar):
    row = edge_index[0]
    col = edge_index[1]
    mask2d = mask.reshape(_N, 1)

    col2 = jnp.concatenate([col, col + _N])
    ewr = jnp.tile(edge_weight[:, None], (1, 16))
    zeros128 = jnp.zeros((_ZROWS, _HID // 2), jnp.float32)

    w0t = jnp.concatenate([W0[:, :_HID // 2], W0[:, _HID // 2:]], axis=0)
    xwt = _spmm_plain(w0t, col2, row, ewr, zeros128)
    hpt = _spmm_w256(xwt, col2, row, ewr, zeros128)

    wcat = jnp.concatenate([Wmu, Wlogvar], axis=1)
    hw = _enc_matmul(hpt.reshape(2, _N, _HID // 2), wcat)
    zcat2 = _spmm_edge(hw, col, row, ewr, zeros128)

    zfull, sums = _prep(zcat2.reshape(2, _N, 2 * _LAT), eps, mask2d,
                        W0, Wmu, Wlogvar)

    tpart = _edge_dot(zfull, row, col)
    s1 = _decoder_sum(zfull, mask2d)
    loss = _finalize(sums, s1, tpart.reshape(4, 128))
    return loss[0, 0]


# speedup vs baseline: 4.5260x; 1.0006x over previous
import functools

import jax
import jax.numpy as jnp
from jax import lax
from jax.experimental import pallas as pl
from jax.experimental.pallas import tpu as pltpu
from jax.experimental.pallas import tpu_sc as plsc

_N = 10000
_E = 160000
_HID = 256
_LAT = 64

_RB = 400
_BM = 400
_BN = 2000

_NSUB = 16
_CH = 80

_ROW_CHUNKS = [(632 * k, 632) for k in range(15)] + [(632 * 15, _N - 632 * 15)]
_ZROWS = 632


def _make_spmm(weighted, colsplit):
    width = _HID // 2
    epw = _E // _NSUB if colsplit else _E // (2 * _NSUB)
    nloop = epw // _CH - (1 if epw % _CH == 0 else 0)
    tail = epw - nloop * _CH
    mesh = plsc.VectorSubcoreMesh(core_axis_name="c", subcore_axis_name="s")

    half = _CH // 2
    scratch = [
        pltpu.VMEM((_CH,), jnp.int32),
        pltpu.VMEM((_CH,), jnp.int32),
        pltpu.VMEM((tail,), jnp.int32),
        pltpu.VMEM((half,), jnp.int32),
        pltpu.VMEM((half,), jnp.int32),
        pltpu.VMEM((half,), jnp.int32),
        pltpu.VMEM((half,), jnp.int32),
        pltpu.VMEM((_CH, width), jnp.float32),
        pltpu.VMEM((_CH, width), jnp.float32),
        pltpu.VMEM((_CH, 16), jnp.float32),
        pltpu.VMEM((_CH, 16), jnp.float32),
        pltpu.VMEM_SHARED((_N, width), jnp.float32),
        pltpu.SemaphoreType.DMA,
        pltpu.SemaphoreType.DMA,
        pltpu.SemaphoreType.DMA,
        pltpu.SemaphoreType.DMA,
        pltpu.SemaphoreType.DMA,
        pltpu.SemaphoreType.DMA,
        pltpu.SemaphoreType.DMA,
        pltpu.SemaphoreType.DMA,
    ]

    def body(table, gidx, rowi, ewr, zeros, out,
             gi0, gi1, git, si0a, si0b, si1a, si1b, r0, r1, w0, w1, acc,
             sg0, sg1, ss0, ss1, sw0, sw1, sc0, sc1):
        c = lax.axis_index("c")
        s = lax.axis_index("s")
        if colsplit:
            goff = c * _E + s * epw
            soff = s * epw
        else:
            goff = (s * 2 + c) * epw
            soff = goff

        gibufs = (gi0, gi1)
        rbufs = (r0, r1)
        sibufs = ((si0a, si0b), (si1a, si1b))
        wbufs = (w0, w1)
        sgs = (sg0, sg1)
        sss = (ss0, ss1)
        sws = (sw0, sw1)
        scs = (sc0, sc1)

        for cs, (off, sz) in enumerate(_ROW_CHUNKS):
            @pl.when(s == cs)
            def _(off=off, sz=sz):
                pltpu.sync_copy(zeros.at[pl.ds(0, sz)], acc.at[pl.ds(off, sz)])
        plsc.subcore_barrier()

        def idx_descs(k, p):
            descs = [(gidx.at[pl.ds(goff + k * _CH, _CH)], gibufs[p], sgs[p]),
                     (rowi.at[pl.ds(soff + k * _CH, half)],
                      sibufs[p][0], sss[p]),
                     (rowi.at[pl.ds(soff + k * _CH + half, half)],
                      sibufs[p][1], sss[p])]
            if weighted:
                descs.append(
                    (ewr.at[pl.ds(soff + k * _CH, _CH)], wbufs[p], sws[p]))
            return descs

        def issue_idx(k, p):
            for src, dst, sem in idx_descs(k, p):
                pltpu.async_copy(src, dst, sem)

        def wait_idx(k, p):
            for src, dst, sem in idx_descs(k, p):
                pltpu.make_async_copy(src, dst, sem).wait()

        def gather_desc(p):
            return (table.at[gibufs[p]], rbufs[p], sgs[p])

        def scale(p, lo, hi):
            rb = rbufs[p]
            wb = wbufs[p]
            for i in range(lo, hi):
                w = wb[i, :]
                for j in range(width // 16):
                    rb[i, 16 * j:16 * (j + 1)] = rb[i, 16 * j:16 * (j + 1)] * w

        def step(k, p):
            src, dst, sem = gather_desc(p)
            pltpu.make_async_copy(src, dst, sem).wait()

            @pl.when(k + 1 < nloop)
            def _():
                wait_idx(k + 1, 1 - p)
                s2, d2, m2 = gather_desc(1 - p)
                pltpu.async_copy(s2, d2, m2)

            if weighted:
                scale(p, 0, half)
                pltpu.async_copy(rbufs[p].at[pl.ds(0, half)],
                                 acc.at[sibufs[p][0]], scs[p], add=True)
                scale(p, half, _CH)
                pltpu.sync_copy(rbufs[p].at[pl.ds(half, half)],
                                acc.at[sibufs[p][1]], add=True)
                pltpu.make_async_copy(rbufs[p].at[pl.ds(0, half)],
                                      acc.at[sibufs[p][0]], scs[p]).wait()
            else:
                pltpu.async_copy(rbufs[p].at[pl.ds(0, half)],
                                 acc.at[sibufs[p][0]], scs[p], add=True)
                pltpu.sync_copy(rbufs[p].at[pl.ds(half, half)],
                                acc.at[sibufs[p][1]], add=True)
                pltpu.make_async_copy(rbufs[p].at[pl.ds(0, half)],
                                      acc.at[sibufs[p][0]], scs[p]).wait()

            @pl.when(k + 2 < nloop)
            def _():
                issue_idx(k + 2, p)

        issue_idx(0, 0)
        issue_idx(1, 1)
        wait_idx(0, 0)
        src0, dst0, sem0 = gather_desc(0)
        pltpu.async_copy(src0, dst0, sem0)

        def it(k, carry):
            @pl.when(k % 2 == 0)
            def _():
                step(k, 0)

            @pl.when(k % 2 == 1)
            def _():
                step(k, 1)

            return carry

        lax.fori_loop(0, nloop, it, jnp.int32(0))

        p = nloop % 2
        nh = tail // half
        tdescs = [(gidx.at[pl.ds(goff + nloop * _CH, tail)], git, sgs[p])]
        for h in range(nh):
            tdescs.append(
                (rowi.at[pl.ds(soff + nloop * _CH + h * half, half)],
                 sibufs[p][h], sss[p]))
        if weighted:
            tdescs.append((ewr.at[pl.ds(soff + nloop * _CH, tail)],
                           wbufs[p].at[pl.ds(0, tail)], sws[p]))
        for src, dst, sem in tdescs:
            pltpu.async_copy(src, dst, sem)
        for src, dst, sem in tdescs:
            pltpu.make_async_copy(src, dst, sem).wait()
        pltpu.async_copy(table.at[git], rbufs[p].at[pl.ds(0, tail)], sgs[p])
        pltpu.make_async_copy(table.at[git], rbufs[p].at[pl.ds(0, tail)],
                              sgs[p]).wait()
        if weighted:
            scale(p, 0, tail)
        for h in range(nh):
            pltpu.sync_copy(rbufs[p].at[pl.ds(h * half, half)],
                            acc.at[sibufs[p][h]], add=True)

        plsc.subcore_barrier()
        for cs, (off, sz) in enumerate(_ROW_CHUNKS):
            @pl.when(s == cs)
            def _(off=off, sz=sz):
                pltpu.sync_copy(acc.at[pl.ds(off, sz)],
                                out.at[pl.ds(c * _N + off, sz)])

    return functools.partial(
        pl.kernel, body,
        out_type=jax.ShapeDtypeStruct((2 * _N, width), jnp.float32),
        mesh=mesh, scratch_types=scratch)()


def _make_edge_dot():
    nwork = 2 * _NSUB
    epw = _E // nwork
    nloop = epw // _CH - (1 if epw % _CH == 0 else 0)
    tail = epw - nloop * _CH
    mesh = plsc.VectorSubcoreMesh(core_axis_name="c", subcore_axis_name="s")

    scratch = [
        pltpu.VMEM((epw,), jnp.int32),
        pltpu.VMEM((epw,), jnp.int32),
        pltpu.VMEM((_CH, 2 * _LAT), jnp.float32),
        pltpu.VMEM((_CH, 2 * _LAT), jnp.float32),
        pltpu.VMEM((_CH, 2 * _LAT), jnp.float32),
        pltpu.VMEM((_CH, 2 * _LAT), jnp.float32),
        pltpu.VMEM((16,), jnp.float32),
        pltpu.SemaphoreType.DMA,
        pltpu.SemaphoreType.DMA,
        pltpu.SemaphoreType.DMA,
        pltpu.SemaphoreType.DMA,
    ]

    def body(zfull, rowi, coli, out,
             gr, gc, za0, za1, zb0, zb1, accv, sa0, sa1, sb0, sb1):
        c = lax.axis_index("c")
        s = lax.axis_index("s")
        wid = s * 2 + c
        off = wid * epw

        zas = (za0, za1)
        zbs = (zb0, zb1)
        sas = (sa0, sa1)
        sbs = (sb0, sb1)

        pltpu.sync_copy(rowi.at[pl.ds(off, epw)], gr)
        pltpu.sync_copy(coli.at[pl.ds(off, epw)], gc)
        accv[...] = jnp.zeros((16,), jnp.float32)

        def gather_descs(k, p, n):
            return [
                (zfull.at[gr.at[pl.ds(k * _CH, n)]],
                 zas[p].at[pl.ds(0, n)], sas[p]),
                (zfull.at[gc.at[pl.ds(k * _CH, n)]],
                 zbs[p].at[pl.ds(0, n)], sbs[p]),
            ]

        def issue(k, p, n):
            for src, dst, sem in gather_descs(k, p, n):
                pltpu.async_copy(src, dst, sem)

        def wait(k, p, n):
            for src, dst, sem in gather_descs(k, p, n):
                pltpu.make_async_copy(src, dst, sem).wait()

        def accum(p, n):
            parts = [jnp.zeros((16,), jnp.float32) for _ in range(4)]
            for i in range(n):
                for j in range(_LAT // 16):
                    parts[j] = parts[j] + (
                        zas[p][i, _LAT + 16 * j:_LAT + 16 * (j + 1)]
                        * zbs[p][i, 16 * j:16 * (j + 1)])
            accv[...] = accv[...] + ((parts[0] + parts[1])
                                     + (parts[2] + parts[3]))

        def step(k, p):
            @pl.when(k + 1 < nloop)
            def _():
                issue(k + 1, 1 - p, _CH)

            wait(k, p, _CH)
            accum(p, _CH)

        issue(0, 0, _CH)

        def it(k, carry):
            @pl.when(k % 2 == 0)
            def _():
                step(k, 0)

            @pl.when(k % 2 == 1)
            def _():
                step(k, 1)

            return carry

        lax.fori_loop(0, nloop, it, jnp.int32(0))

        p = nloop % 2
        issue(nloop, p, tail)
        wait(nloop, p, tail)
        accum(p, tail)

        pltpu.sync_copy(accv, out.at[pl.ds(wid * 16, 16)])

    return functools.partial(
        pl.kernel, body,
        out_type=jax.ShapeDtypeStruct((nwork * 16,), jnp.float32),
        mesh=mesh, scratch_types=scratch)()


def _mm_body(hp_ref, w_ref, o_ref):
    h = jnp.maximum(jnp.concatenate([hp_ref[0], hp_ref[1]], axis=1), 0.0)
    o_ref[...] = jnp.dot(h, w_ref[...], preferred_element_type=jnp.float32)


def _enc_matmul(hp3, wcat):
    grid = (_N // _RB,)
    return pl.pallas_call(
        _mm_body,
        grid=grid,
        in_specs=[
            pl.BlockSpec((2, _RB, _HID // 2), lambda i: (0, i, 0)),
            pl.BlockSpec((_HID, 2 * _LAT), lambda i: (0, 0)),
        ],
        out_specs=pl.BlockSpec((_RB, 2 * _LAT), lambda i: (i, 0)),
        out_shape=jax.ShapeDtypeStruct((_N, 2 * _LAT), jnp.float32),
    )(hp3, wcat)


def _prep_body(zc_ref, eps_ref, mask_ref, w0_ref, wmu_ref, wlv_ref,
               zfull_ref, sums_ref):
    i = pl.program_id(0)
    zcat = zc_ref[0] + zc_ref[1]
    zm = zcat[:, :_LAT]
    zlv = zcat[:, _LAT:]
    z = zm + eps_ref[...] * jnp.exp(0.5 * zlv)
    mask = mask_ref[...]
    zfull_ref[...] = jnp.concatenate([z, mask * z], axis=1)

    kl_part = jnp.sum(zlv - zm * zm - jnp.exp(zlv) + 1.0)
    l2_w0 = jnp.sum(w0_ref[...] * w0_ref[...])
    l2_w = jnp.sum(wmu_ref[...] * wmu_ref[...]) + jnp.sum(wlv_ref[...] * wlv_ref[...])
    msum = jnp.sum(mask)

    lane = lax.broadcasted_iota(jnp.int32, (1, 128), 1)
    vec = (jnp.where(lane == 0, kl_part, 0.0)
           + jnp.where(lane == 1, l2_w0, 0.0)
           + jnp.where((lane == 2) & (i == 0), l2_w, 0.0)
           + jnp.where(lane == 3, msum, 0.0))

    @pl.when(i == 0)
    def _():
        sums_ref[...] = jnp.zeros_like(sums_ref)

    sums_ref[...] += vec


def _prep(zcat3, eps, mask2d, w0, wmu, wlv):
    grid = (_N // _RB,)
    return pl.pallas_call(
        _prep_body,
        grid=grid,
        in_specs=[
            pl.BlockSpec((2, _RB, 2 * _LAT), lambda i: (0, i, 0)),
            pl.BlockSpec((_RB, _LAT), lambda i: (i, 0)),
            pl.BlockSpec((_RB, 1), lambda i: (i, 0)),
            pl.BlockSpec((_RB, _HID), lambda i: (i, 0)),
            pl.BlockSpec((_HID, _LAT), lambda i: (0, 0)),
            pl.BlockSpec((_HID, _LAT), lambda i: (0, 0)),
        ],
        out_specs=[
            pl.BlockSpec((_RB, 2 * _LAT), lambda i: (i, 0)),
            pl.BlockSpec((1, 128), lambda i: (0, 0)),
        ],
        out_shape=[
            jax.ShapeDtypeStruct((_N, 2 * _LAT), jnp.float32),
            jax.ShapeDtypeStruct((1, 128), jnp.float32),
        ],
    )(zcat3, eps, mask2d, w0, wmu, wlv)


def _dec_body(zi_ref, zj_ref, mask_ref, out_ref):
    i = pl.program_id(0)
    j = pl.program_id(1)
    logits = lax.dot_general(zi_ref[:, :_LAT], zj_ref[:, :_LAT],
                             (((1,), (1,)), ((), ())),
                             preferred_element_type=jnp.float32)
    sp = jnp.maximum(logits, 0.0) + jnp.log1p(jnp.exp(-jnp.abs(logits)))
    part = jnp.sum(sp * mask_ref[...])

    @pl.when((i == 0) & (j == 0))
    def _():
        out_ref[0, 0] = 0.0

    out_ref[0, 0] += part


def _decoder_sum(z, mask2d):
    grid = (_N // _BM, _N // _BN)
    return pl.pallas_call(
        _dec_body,
        grid=grid,
        in_specs=[
            pl.BlockSpec((_BM, 2 * _LAT), lambda i, j: (i, 0)),
            pl.BlockSpec((_BN, 2 * _LAT), lambda i, j: (j, 0)),
            pl.BlockSpec((_BM, 1), lambda i, j: (i, 0)),
        ],
        out_specs=pl.BlockSpec(memory_space=pltpu.SMEM),
        out_shape=jax.ShapeDtypeStruct((1, 1), jnp.float32),
    )(z, z, mask2d)


def _fin_body(sums_ref, s1_ref, tp_ref, out_ref):
    t = jnp.sum(tp_ref[...])
    kl_sum = sums_ref[0, 0]
    l2 = 0.5 * (sums_ref[0, 1] + sums_ref[0, 2])
    msum = sums_ref[0, 3]
    kl = -0.5 * kl_sum / (_N * _LAT)
    masked_ce = (s1_ref[0, 0] - t) / (_N * _N) / (msum / _N)
    out_ref[0, 0] = l2 + masked_ce + kl


def _finalize(sums, s1, tpart):
    return pl.pallas_call(
        _fin_body,
        in_specs=[
            pl.BlockSpec((1, 128), lambda: (0, 0)),
            pl.BlockSpec(memory_space=pltpu.SMEM),
            pl.BlockSpec((4, 128), lambda: (0, 0)),
        ],
        out_specs=pl.BlockSpec(memory_space=pltpu.SMEM),
        out_shape=jax.ShapeDtypeStruct((1, 1), jnp.float32),
    )(sums, s1, tpart)


_spmm_plain = _make_spmm(weighted=False, colsplit=True)
_spmm_w256 = _make_spmm(weighted=True, colsplit=True)
_spmm_edge = _make_spmm(weighted=True, colsplit=False)
_edge_dot = _make_edge_dot()


def kernel(edge_index, edge_weight, mask, eps, W0, Wmu, Wlogvar):
    row = edge_index[0]
    col = edge_index[1]
    mask2d = mask.reshape(_N, 1)

    col2 = jnp.concatenate([col, col + _N])
    ewr = jnp.tile(edge_weight[:, None], (1, 16))
    zeros128 = jnp.zeros((_ZROWS, _HID // 2), jnp.float32)

    w0t = jnp.concatenate([W0[:, :_HID // 2], W0[:, _HID // 2:]], axis=0)
    xwt = _spmm_plain(w0t, col2, row, ewr, zeros128)
    hpt = _spmm_w256(xwt, col2, row, ewr, zeros128)

    wcat = jnp.concatenate([Wmu, Wlogvar], axis=1)
    hw = _enc_matmul(hpt.reshape(2, _N, _HID // 2), wcat)
    zcat2 = _spmm_edge(hw, col, row, ewr, zeros128)

    zfull, sums = _prep(zcat2.reshape(2, _N, 2 * _LAT), eps, mask2d,
                        W0, Wmu, Wlogvar)

    s1 = _decoder_sum(zfull, mask2d)
    tpart = _edge_dot(zfull, row, col)
    loss = _finalize(sums, s1, tpart.reshape(4, 128))
    return loss[0, 0]


# speedup vs baseline: 4.8115x; 1.0631x over previous
import functools

import jax
import jax.numpy as jnp
from jax import lax
from jax.experimental import pallas as pl
from jax.experimental.pallas import tpu as pltpu
from jax.experimental.pallas import tpu_sc as plsc

_N = 10000
_E = 160000
_HID = 256
_LAT = 64

_RB = 400
_BM = 400
_BN = 2000

_NSUB = 16
_CH = 80

_ROW_CHUNKS = [(632 * k, 632) for k in range(15)] + [(632 * 15, _N - 632 * 15)]
_ZROWS = 632


def _make_spmm(weighted, colsplit):
    width = _HID // 2
    epw = _E // _NSUB if colsplit else _E // (2 * _NSUB)
    nloop = epw // _CH - (1 if epw % _CH == 0 else 0)
    tail = epw - nloop * _CH
    mesh = plsc.VectorSubcoreMesh(core_axis_name="c", subcore_axis_name="s")

    half = _CH // 2
    scratch = [
        pltpu.VMEM((_CH,), jnp.int32),
        pltpu.VMEM((_CH,), jnp.int32),
        pltpu.VMEM((tail,), jnp.int32),
        pltpu.VMEM((half,), jnp.int32),
        pltpu.VMEM((half,), jnp.int32),
        pltpu.VMEM((half,), jnp.int32),
        pltpu.VMEM((half,), jnp.int32),
        pltpu.VMEM((tail,), jnp.int32),
        pltpu.VMEM((_CH, width), jnp.float32),
        pltpu.VMEM((_CH, width), jnp.float32),
        pltpu.VMEM((_CH, 16), jnp.float32),
        pltpu.VMEM((_CH, 16), jnp.float32),
        pltpu.VMEM_SHARED((_N, width), jnp.float32),
        pltpu.SemaphoreType.DMA,
        pltpu.SemaphoreType.DMA,
        pltpu.SemaphoreType.DMA,
        pltpu.SemaphoreType.DMA,
        pltpu.SemaphoreType.DMA,
        pltpu.SemaphoreType.DMA,
        pltpu.SemaphoreType.DMA,
        pltpu.SemaphoreType.DMA,
    ]

    def body(table, gidx, rowi, ewr, zeros, out,
             gi0, gi1, git, si0a, si0b, si1a, si1b, sit, r0, r1, w0, w1, acc,
             sg0, sg1, ss0, ss1, sw0, sw1, sc0, sc1):
        c = lax.axis_index("c")
        s = lax.axis_index("s")
        if colsplit:
            goff = c * _E + s * epw
            soff = s * epw
        else:
            goff = (s * 2 + c) * epw
            soff = goff

        gibufs = (gi0, gi1)
        rbufs = (r0, r1)
        sibufs = ((si0a, si0b), (si1a, si1b))
        wbufs = (w0, w1)
        sgs = (sg0, sg1)
        sss = (ss0, ss1)
        sws = (sw0, sw1)
        scs = (sc0, sc1)

        for cs, (off, sz) in enumerate(_ROW_CHUNKS):
            @pl.when(s == cs)
            def _(off=off, sz=sz):
                pltpu.sync_copy(zeros.at[pl.ds(0, sz)], acc.at[pl.ds(off, sz)])
        plsc.subcore_barrier()

        def idx_descs(k, p):
            descs = [(gidx.at[pl.ds(goff + k * _CH, _CH)], gibufs[p], sgs[p]),
                     (rowi.at[pl.ds(soff + k * _CH, half)],
                      sibufs[p][0], sss[p]),
                     (rowi.at[pl.ds(soff + k * _CH + half, half)],
                      sibufs[p][1], sss[p])]
            if weighted:
                descs.append(
                    (ewr.at[pl.ds(soff + k * _CH, _CH)], wbufs[p], sws[p]))
            return descs

        def issue_idx(k, p):
            for src, dst, sem in idx_descs(k, p):
                pltpu.async_copy(src, dst, sem)

        def wait_idx(k, p):
            for src, dst, sem in idx_descs(k, p):
                pltpu.make_async_copy(src, dst, sem).wait()

        def gather_desc(p):
            return (table.at[gibufs[p]], rbufs[p], sgs[p])

        def scale(p, lo, hi):
            rb = rbufs[p]
            wb = wbufs[p]
            for i in range(lo, hi):
                w = wb[i, :]
                for j in range(width // 16):
                    rb[i, 16 * j:16 * (j + 1)] = rb[i, 16 * j:16 * (j + 1)] * w

        def step(k, p):
            src, dst, sem = gather_desc(p)
            pltpu.make_async_copy(src, dst, sem).wait()

            @pl.when(k + 1 < nloop)
            def _():
                wait_idx(k + 1, 1 - p)
                s2, d2, m2 = gather_desc(1 - p)
                pltpu.async_copy(s2, d2, m2)

            if weighted:
                scale(p, 0, half)
                pltpu.async_copy(rbufs[p].at[pl.ds(0, half)],
                                 acc.at[sibufs[p][0]], scs[p], add=True)
                scale(p, half, _CH)
                pltpu.sync_copy(rbufs[p].at[pl.ds(half, half)],
                                acc.at[sibufs[p][1]], add=True)
                pltpu.make_async_copy(rbufs[p].at[pl.ds(0, half)],
                                      acc.at[sibufs[p][0]], scs[p]).wait()
            else:
                pltpu.async_copy(rbufs[p].at[pl.ds(0, half)],
                                 acc.at[sibufs[p][0]], scs[p], add=True)
                pltpu.sync_copy(rbufs[p].at[pl.ds(half, half)],
                                acc.at[sibufs[p][1]], add=True)
                pltpu.make_async_copy(rbufs[p].at[pl.ds(0, half)],
                                      acc.at[sibufs[p][0]], scs[p]).wait()

            @pl.when(k + 2 < nloop)
            def _():
                issue_idx(k + 2, p)

        issue_idx(0, 0)
        issue_idx(1, 1)
        wait_idx(0, 0)
        src0, dst0, sem0 = gather_desc(0)
        pltpu.async_copy(src0, dst0, sem0)

        def it(k, carry):
            @pl.when(k % 2 == 0)
            def _():
                step(k, 0)

            @pl.when(k % 2 == 1)
            def _():
                step(k, 1)

            return carry

        lax.fori_loop(0, nloop, it, jnp.int32(0))

        p = nloop % 2
        tdescs = [(gidx.at[pl.ds(goff + nloop * _CH, tail)], git, sgs[p]),
                  (rowi.at[pl.ds(soff + nloop * _CH, tail)], sit, sss[p])]
        if weighted:
            tdescs.append((ewr.at[pl.ds(soff + nloop * _CH, tail)],
                           wbufs[p].at[pl.ds(0, tail)], sws[p]))
        for src, dst, sem in tdescs:
            pltpu.async_copy(src, dst, sem)
        for src, dst, sem in tdescs:
            pltpu.make_async_copy(src, dst, sem).wait()
        pltpu.async_copy(table.at[git], rbufs[p].at[pl.ds(0, tail)], sgs[p])
        pltpu.make_async_copy(table.at[git], rbufs[p].at[pl.ds(0, tail)],
                              sgs[p]).wait()
        if weighted:
            scale(p, 0, tail)
        pltpu.sync_copy(rbufs[p].at[pl.ds(0, tail)], acc.at[sit], add=True)

        plsc.subcore_barrier()
        for cs, (off, sz) in enumerate(_ROW_CHUNKS):
            @pl.when(s == cs)
            def _(off=off, sz=sz):
                pltpu.sync_copy(acc.at[pl.ds(off, sz)],
                                out.at[pl.ds(c * _N + off, sz)])

    return functools.partial(
        pl.kernel, body,
        out_type=jax.ShapeDtypeStruct((2 * _N, width), jnp.float32),
        mesh=mesh, scratch_types=scratch)()


def _make_edge_dot():
    nwork = 2 * _NSUB
    epw = _E // nwork
    nloop = epw // _CH - (1 if epw % _CH == 0 else 0)
    tail = epw - nloop * _CH
    mesh = plsc.VectorSubcoreMesh(core_axis_name="c", subcore_axis_name="s")

    scratch = [
        pltpu.VMEM((epw,), jnp.int32),
        pltpu.VMEM((epw,), jnp.int32),
        pltpu.VMEM((_CH, 2 * _LAT), jnp.float32),
        pltpu.VMEM((_CH, 2 * _LAT), jnp.float32),
        pltpu.VMEM((_CH, 2 * _LAT), jnp.float32),
        pltpu.VMEM((_CH, 2 * _LAT), jnp.float32),
        pltpu.VMEM((16,), jnp.float32),
        pltpu.SemaphoreType.DMA,
        pltpu.SemaphoreType.DMA,
        pltpu.SemaphoreType.DMA,
        pltpu.SemaphoreType.DMA,
    ]

    def body(zfull, rowi, coli, out,
             gr, gc, za0, za1, zb0, zb1, accv, sa0, sa1, sb0, sb1):
        c = lax.axis_index("c")
        s = lax.axis_index("s")
        wid = s * 2 + c
        off = wid * epw

        zas = (za0, za1)
        zbs = (zb0, zb1)
        sas = (sa0, sa1)
        sbs = (sb0, sb1)

        pltpu.sync_copy(rowi.at[pl.ds(off, epw)], gr)
        pltpu.sync_copy(coli.at[pl.ds(off, epw)], gc)
        accv[...] = jnp.zeros((16,), jnp.float32)

        def gather_descs(k, p, n):
            return [
                (zfull.at[gr.at[pl.ds(k * _CH, n)]],
                 zas[p].at[pl.ds(0, n)], sas[p]),
                (zfull.at[gc.at[pl.ds(k * _CH, n)]],
                 zbs[p].at[pl.ds(0, n)], sbs[p]),
            ]

        def issue(k, p, n):
            for src, dst, sem in gather_descs(k, p, n):
                pltpu.async_copy(src, dst, sem)

        def wait(k, p, n):
            for src, dst, sem in gather_descs(k, p, n):
                pltpu.make_async_copy(src, dst, sem).wait()

        def accum(p, n):
            parts = [jnp.zeros((16,), jnp.float32) for _ in range(4)]
            for i in range(n):
                for j in range(_LAT // 16):
                    parts[j] = parts[j] + (
                        zas[p][i, _LAT + 16 * j:_LAT + 16 * (j + 1)]
                        * zbs[p][i, 16 * j:16 * (j + 1)])
            accv[...] = accv[...] + ((parts[0] + parts[1])
                                     + (parts[2] + parts[3]))

        def step(k, p):
            @pl.when(k + 1 < nloop)
            def _():
                issue(k + 1, 1 - p, _CH)

            wait(k, p, _CH)
            accum(p, _CH)

        issue(0, 0, _CH)

        def it(k, carry):
            @pl.when(k % 2 == 0)
            def _():
                step(k, 0)

            @pl.when(k % 2 == 1)
            def _():
                step(k, 1)

            return carry

        lax.fori_loop(0, nloop, it, jnp.int32(0))

        p = nloop % 2
        issue(nloop, p, tail)
        wait(nloop, p, tail)
        accum(p, tail)

        pltpu.sync_copy(accv, out.at[pl.ds(wid * 16, 16)])

    return functools.partial(
        pl.kernel, body,
        out_type=jax.ShapeDtypeStruct((nwork * 16,), jnp.float32),
        mesh=mesh, scratch_types=scratch)()


def _mm_body(hp_ref, w_ref, o_ref):
    h = jnp.maximum(jnp.concatenate([hp_ref[0], hp_ref[1]], axis=1), 0.0)
    o_ref[...] = jnp.dot(h, w_ref[...], preferred_element_type=jnp.float32)


def _enc_matmul(hp3, wcat):
    grid = (_N // _RB,)
    return pl.pallas_call(
        _mm_body,
        grid=grid,
        in_specs=[
            pl.BlockSpec((2, _RB, _HID // 2), lambda i: (0, i, 0)),
            pl.BlockSpec((_HID, 2 * _LAT), lambda i: (0, 0)),
        ],
        out_specs=pl.BlockSpec((_RB, 2 * _LAT), lambda i: (i, 0)),
        out_shape=jax.ShapeDtypeStruct((_N, 2 * _LAT), jnp.float32),
    )(hp3, wcat)


def _prep_body(zc_ref, eps_ref, mask_ref, w0_ref, wmu_ref, wlv_ref,
               zfull_ref, sums_ref):
    i = pl.program_id(0)
    zcat = zc_ref[0] + zc_ref[1]
    zm = zcat[:, :_LAT]
    zlv = zcat[:, _LAT:]
    z = zm + eps_ref[...] * jnp.exp(0.5 * zlv)
    mask = mask_ref[...]
    zfull_ref[...] = jnp.concatenate([z, mask * z], axis=1)

    kl_part = jnp.sum(zlv - zm * zm - jnp.exp(zlv) + 1.0)
    l2_w0 = jnp.sum(w0_ref[...] * w0_ref[...])
    l2_w = jnp.sum(wmu_ref[...] * wmu_ref[...]) + jnp.sum(wlv_ref[...] * wlv_ref[...])
    msum = jnp.sum(mask)

    lane = lax.broadcasted_iota(jnp.int32, (1, 128), 1)
    vec = (jnp.where(lane == 0, kl_part, 0.0)
           + jnp.where(lane == 1, l2_w0, 0.0)
           + jnp.where((lane == 2) & (i == 0), l2_w, 0.0)
           + jnp.where(lane == 3, msum, 0.0))

    @pl.when(i == 0)
    def _():
        sums_ref[...] = jnp.zeros_like(sums_ref)

    sums_ref[...] += vec


def _prep(zcat3, eps, mask2d, w0, wmu, wlv):
    grid = (_N // _RB,)
    return pl.pallas_call(
        _prep_body,
        grid=grid,
        in_specs=[
            pl.BlockSpec((2, _RB, 2 * _LAT), lambda i: (0, i, 0)),
            pl.BlockSpec((_RB, _LAT), lambda i: (i, 0)),
            pl.BlockSpec((_RB, 1), lambda i: (i, 0)),
            pl.BlockSpec((_RB, _HID), lambda i: (i, 0)),
            pl.BlockSpec((_HID, _LAT), lambda i: (0, 0)),
            pl.BlockSpec((_HID, _LAT), lambda i: (0, 0)),
        ],
        out_specs=[
            pl.BlockSpec((_RB, 2 * _LAT), lambda i: (i, 0)),
            pl.BlockSpec((1, 128), lambda i: (0, 0)),
        ],
        out_shape=[
            jax.ShapeDtypeStruct((_N, 2 * _LAT), jnp.float32),
            jax.ShapeDtypeStruct((1, 128), jnp.float32),
        ],
    )(zcat3, eps, mask2d, w0, wmu, wlv)


def _dec_body(zi_ref, zj_ref, mask_ref, out_ref):
    i = pl.program_id(0)
    j = pl.program_id(1)
    logits = lax.dot_general(zi_ref[:, :_LAT], zj_ref[:, :_LAT],
                             (((1,), (1,)), ((), ())),
                             preferred_element_type=jnp.float32)
    l16 = logits.astype(jnp.bfloat16)
    sp = (jnp.maximum(l16, jnp.bfloat16(0.0))
          + jnp.log1p(jnp.exp(-jnp.abs(l16))))
    m16 = mask_ref[...].astype(jnp.bfloat16)
    colsum = lax.dot_general(sp, m16, (((0,), (0,)), ((), ())),
                             preferred_element_type=jnp.float32)
    part = jnp.sum(colsum)

    @pl.when((i == 0) & (j == 0))
    def _():
        out_ref[0, 0] = 0.0

    out_ref[0, 0] += part


def _decoder_sum(z, mask2d):
    grid = (_N // _BM, _N // _BN)
    return pl.pallas_call(
        _dec_body,
        grid=grid,
        in_specs=[
            pl.BlockSpec((_BM, 2 * _LAT), lambda i, j: (i, 0)),
            pl.BlockSpec((_BN, 2 * _LAT), lambda i, j: (j, 0)),
            pl.BlockSpec((_BM, 1), lambda i, j: (i, 0)),
        ],
        out_specs=pl.BlockSpec(memory_space=pltpu.SMEM),
        out_shape=jax.ShapeDtypeStruct((1, 1), jnp.float32),
    )(z, z, mask2d)


def _fin_body(sums_ref, s1_ref, tp_ref, out_ref):
    t = jnp.sum(tp_ref[...])
    kl_sum = sums_ref[0, 0]
    l2 = 0.5 * (sums_ref[0, 1] + sums_ref[0, 2])
    msum = sums_ref[0, 3]
    kl = -0.5 * kl_sum / (_N * _LAT)
    masked_ce = (s1_ref[0, 0] - t) / (_N * _N) / (msum / _N)
    out_ref[0, 0] = l2 + masked_ce + kl


def _finalize(sums, s1, tpart):
    return pl.pallas_call(
        _fin_body,
        in_specs=[
            pl.BlockSpec((1, 128), lambda: (0, 0)),
            pl.BlockSpec(memory_space=pltpu.SMEM),
            pl.BlockSpec((4, 128), lambda: (0, 0)),
        ],
        out_specs=pl.BlockSpec(memory_space=pltpu.SMEM),
        out_shape=jax.ShapeDtypeStruct((1, 1), jnp.float32),
    )(sums, s1, tpart)


_spmm_plain = _make_spmm(weighted=False, colsplit=True)
_spmm_w256 = _make_spmm(weighted=True, colsplit=True)
_spmm_edge = _make_spmm(weighted=True, colsplit=False)
_edge_dot = _make_edge_dot()


def kernel(edge_index, edge_weight, mask, eps, W0, Wmu, Wlogvar):
    row = edge_index[0]
    col = edge_index[1]
    mask2d = mask.reshape(_N, 1)

    col2 = jnp.concatenate([col, col + _N])
    ewr = jnp.tile(edge_weight[:, None], (1, 16))
    zeros128 = jnp.zeros((_ZROWS, _HID // 2), jnp.float32)

    w0t = jnp.concatenate([W0[:, :_HID // 2], W0[:, _HID // 2:]], axis=0)
    xwt = _spmm_plain(w0t, col2, row, ewr, zeros128)
    hpt = _spmm_w256(xwt, col2, row, ewr, zeros128)

    wcat = jnp.concatenate([Wmu, Wlogvar], axis=1)
    hw = _enc_matmul(hpt.reshape(2, _N, _HID // 2), wcat)
    zcat2 = _spmm_edge(hw, col, row, ewr, zeros128)

    zfull, sums = _prep(zcat2.reshape(2, _N, 2 * _LAT), eps, mask2d,
                        W0, Wmu, Wlogvar)

    s1 = _decoder_sum(zfull, mask2d)
    tpart = _edge_dot(zfull, row, col)
    loss = _finalize(sums, s1, tpart.reshape(4, 128))
    return loss[0, 0]
